# baseline (speedup 1.0000x reference)
import functools

import jax
import jax.numpy as jnp
from jax import lax
from jax.experimental import pallas as pl
from jax.experimental.pallas import tpu as pltpu
from jax.experimental.pallas import tpu_sc as plsc

EMBED_DIM = 64
SLICE = 128
SLICES_PER_CHUNK = 4
CHUNK = SLICE * SLICES_PER_CHUNK


@functools.lru_cache(maxsize=None)
def _make_gather(batch: int):
    info = plsc.get_sparse_core_info()
    num_workers = info.num_cores * info.num_subcores
    b_per_w = batch // num_workers
    assert b_per_w * num_workers == batch and b_per_w % CHUNK == 0
    n_slices = b_per_w // SLICE
    n_chunks = b_per_w // CHUNK

    mesh = plsc.VectorSubcoreMesh(core_axis_name="c", subcore_axis_name="s")

    @functools.partial(
        pl.kernel,
        out_type=jax.ShapeDtypeStruct((batch, EMBED_DIM), jnp.float32),
        mesh=mesh,
        compiler_params=pltpu.CompilerParams(use_tc_tiling_on_sc=False),
        scratch_types=[
            pltpu.VMEM((n_slices, SLICE), jnp.int32),
            pltpu.VMEM((CHUNK, EMBED_DIM), jnp.float32),
            pltpu.SemaphoreType.DMA,
        ],
    )
    def gather_kernel(idx_hbm, table_hbm, out_hbm, idx_v, rows_v, sem):
        wid = lax.axis_index("s") * info.num_cores + lax.axis_index("c")
        pltpu.sync_copy(idx_hbm.at[wid], idx_v)
        row0 = wid * b_per_w

        @pl.loop(0, n_chunks)
        def _chunk(c):
            copies = []
            for j in range(SLICES_PER_CHUNK):
                s = c * SLICES_PER_CHUNK + j
                copies.append(
                    pltpu.async_copy(
                        table_hbm.at[idx_v.at[s]],
                        rows_v.at[pl.ds(j * SLICE, SLICE)],
                        sem,
                    )
                )
            for cp in copies:
                cp.wait()
            pltpu.sync_copy(rows_v, out_hbm.at[pl.ds(row0 + c * CHUNK, CHUNK)])

    def run(idx_flat, table):
        idx3 = idx_flat.reshape(num_workers, n_slices, SLICE)
        return gather_kernel(idx3, table)

    return run


def kernel(x, embedding):
    b0, b1 = x.shape
    out = _make_gather(b0 * b1)(x.reshape(-1).astype(jnp.int32), embedding)
    return out.reshape(b0, b1, EMBED_DIM)


# speedup vs baseline: 1.0240x; 1.0240x over previous
import functools

import jax
import jax.numpy as jnp
from jax import lax
from jax.experimental import pallas as pl
from jax.experimental.pallas import tpu as pltpu
from jax.experimental.pallas import tpu_sc as plsc

EMBED_DIM = 64
SLICE = 128
SLICES_PER_CHUNK = 2
CHUNK = SLICE * SLICES_PER_CHUNK
NBUF = 4


@functools.lru_cache(maxsize=None)
def _make_gather(batch: int):
    info = plsc.get_sparse_core_info()
    num_workers = info.num_cores * info.num_subcores
    b_per_w = batch // num_workers
    assert b_per_w * num_workers == batch and b_per_w % CHUNK == 0
    n_slices = b_per_w // SLICE
    n_chunks = b_per_w // CHUNK

    mesh = plsc.VectorSubcoreMesh(core_axis_name="c", subcore_axis_name="s")

    @functools.partial(
        pl.kernel,
        out_type=jax.ShapeDtypeStruct((batch, EMBED_DIM), jnp.float32),
        mesh=mesh,
        compiler_params=pltpu.CompilerParams(use_tc_tiling_on_sc=False),
        scratch_types=[
            pltpu.VMEM((n_slices, SLICE), jnp.int32),
            [pltpu.VMEM((CHUNK, EMBED_DIM), jnp.float32) for _ in range(NBUF)],
            [pltpu.SemaphoreType.DMA for _ in range(NBUF)],
            [pltpu.SemaphoreType.DMA for _ in range(NBUF)],
        ],
    )
    def gather_kernel(idx_hbm, table_hbm, out_hbm, idx_v, bufs, gsems, ssems):
        wid = lax.axis_index("s") * info.num_cores + lax.axis_index("c")
        pltpu.sync_copy(idx_hbm.at[wid], idx_v)
        row0 = wid * b_per_w

        def fire_gathers(c, b):
            for j in range(SLICES_PER_CHUNK):
                pltpu.async_copy(
                    table_hbm.at[idx_v.at[c * SLICES_PER_CHUNK + j]],
                    bufs[b].at[pl.ds(j * SLICE, SLICE)],
                    gsems[b],
                )

        def wait_gathers(b):
            pltpu.make_async_copy(
                out_hbm.at[pl.ds(0, CHUNK)], bufs[b], gsems[b]
            ).wait()

        def fire_store(c, b):
            pltpu.async_copy(
                bufs[b], out_hbm.at[pl.ds(row0 + c * CHUNK, CHUNK)], ssems[b]
            )

        def wait_store(b):
            pltpu.make_async_copy(
                bufs[b], out_hbm.at[pl.ds(0, CHUNK)], ssems[b]
            ).wait()

        for b in range(NBUF):
            fire_gathers(b, b)

        @pl.loop(0, n_chunks - NBUF, step=NBUF)
        def _grp(c0):
            for b in range(NBUF):
                c = c0 + b
                wait_gathers(b)
                fire_store(c, b)
                wait_store(b)
                fire_gathers(c + NBUF, b)

        for b in range(NBUF):
            wait_gathers(b)
            fire_store(n_chunks - NBUF + b, b)
        for b in range(NBUF):
            wait_store(b)

    def run(idx_flat, table):
        idx3 = idx_flat.reshape(num_workers, n_slices, SLICE)
        return gather_kernel(idx3, table)

    return run


def kernel(x, embedding):
    b0, b1 = x.shape
    out = _make_gather(b0 * b1)(x.reshape(-1).astype(jnp.int32), embedding)
    return out.reshape(b0, b1, EMBED_DIM)
